# baseline (speedup 1.0000x reference)
import functools

import jax
import jax.numpy as jnp
from jax import lax
from jax.experimental import pallas as pl
from jax.experimental.pallas import tpu as pltpu
from jax.experimental.pallas import tpu_sc as plsc

_VOCAB = 1000000
_D = 64
_SEQ = 200
_BATCH = 1024
_NC = 2
_NS = 16
_NW = _NC * _NS
_ROWS = _BATCH * _SEQ
_CHUNK = 2 * _SEQ
_PAIRS = _CHUNK // 2
_CHUNKS_PER_W = _ROWS // (_NW * _CHUNK)
_HALF = _ROWS // 2
_IDXW = _CHUNKS_PER_W * _PAIRS
_NBUF = 2
_LANES = 16
_VPR = _D // _LANES


def _emb_kernel(x_hbm, tok_hbm, pos2_hbm, out_hbm, pos_v, idx_e, idx_o,
                rows_e, rows_o, pout, sems):
    sem_g, sem_o = sems
    wid = lax.axis_index("s") * _NC + lax.axis_index("c")
    base = wid * _IDXW

    pltpu.sync_copy(pos2_hbm, pos_v)
    pltpu.sync_copy(x_hbm.at[pl.ds(base, _IDXW)], idx_e)
    pltpu.sync_copy(x_hbm.at[pl.ds(_HALF + base, _IDXW)], idx_o)

    def stage(c):
        b = c % _NBUF
        he = pltpu.async_copy(
            tok_hbm.at[idx_e.at[pl.ds(c * _PAIRS, _PAIRS)]],
            rows_e[b],
            sem_g[b],
        )
        ho = pltpu.async_copy(
            tok_hbm.at[idx_o.at[pl.ds(c * _PAIRS, _PAIRS)]],
            rows_o[b],
            sem_g[b],
        )
        return he, ho

    def merge_add(b):
        def pair(p, s2x100):
            p_abs = s2x100 + p
            for j in range(_VPR):
                pout[b][p_abs, pl.ds(j * _LANES, _LANES)] = (
                    rows_e[b][p_abs, pl.ds(j * _LANES, _LANES)]
                    + pos_v[p, pl.ds(j * _LANES, _LANES)]
                )
            for j in range(_VPR):
                pout[b][p_abs, pl.ds(_D + j * _LANES, _LANES)] = (
                    rows_o[b][p_abs, pl.ds(j * _LANES, _LANES)]
                    + pos_v[p, pl.ds(_D + j * _LANES, _LANES)]
                )
            return s2x100

        for s2 in range(2):
            lax.fori_loop(0, _PAIRS // 2, pair, s2 * (_PAIRS // 2), unroll=2)

    g = [None] * _CHUNKS_PER_W
    o = [None] * _CHUNKS_PER_W
    g[0] = stage(0)
    for c in range(_CHUNKS_PER_W):
        b = c % _NBUF
        if c + 1 < _CHUNKS_PER_W:
            g[c + 1] = stage(c + 1)
        g[c][0].wait()
        g[c][1].wait()
        if c >= _NBUF:
            o[c - _NBUF].wait()
        merge_add(b)
        o[c] = pltpu.async_copy(
            pout[b],
            out_hbm.at[pl.ds((base + c * _PAIRS), _PAIRS)],
            sem_o[b],
        )
    for c in range(_CHUNKS_PER_W - _NBUF, _CHUNKS_PER_W):
        o[c].wait()


@jax.jit
def _run(x_de, tok64, pos2):
    mesh = plsc.VectorSubcoreMesh(core_axis_name="c", subcore_axis_name="s")
    f = functools.partial(
        pl.kernel,
        out_type=jax.ShapeDtypeStruct((_ROWS // 2, 2 * _D), jnp.float32),
        mesh=mesh,
        scratch_types=[
            pltpu.VMEM((_SEQ // 2, 2 * _D), jnp.float32),
            pltpu.VMEM((_IDXW,), jnp.int32),
            pltpu.VMEM((_IDXW,), jnp.int32),
            [pltpu.VMEM((_PAIRS, _D), jnp.float32) for _ in range(_NBUF)],
            [pltpu.VMEM((_PAIRS, _D), jnp.float32) for _ in range(_NBUF)],
            [pltpu.VMEM((_PAIRS, 2 * _D), jnp.float32) for _ in range(_NBUF)],
            (
                [pltpu.SemaphoreType.DMA for _ in range(_NBUF)],
                [pltpu.SemaphoreType.DMA for _ in range(_NBUF)],
            ),
        ],
        compiler_params=pltpu.CompilerParams(use_tc_tiling_on_sc=False),
    )(_emb_kernel)
    return f(x_de, tok64, pos2)


_TB = 2048
_GRID = (_VOCAB + _TB - 1) // _TB
_TROWS = _GRID * _TB


def _transpose_body(in_ref, out_ref):
    t = jnp.transpose(in_ref[...])
    out_ref[...] = jnp.concatenate([t[: _TB // 2], t[_TB // 2 :]], axis=1)


@jax.jit
def _relayout(tokT):
    return pl.pallas_call(
        _transpose_body,
        grid=(_GRID,),
        in_specs=[pl.BlockSpec((_D, _TB), lambda g: (0, g))],
        out_specs=pl.BlockSpec((_TB // 2, 2 * _D), lambda g: (g, 0)),
        out_shape=jax.ShapeDtypeStruct((_TROWS // 2, 2 * _D), jnp.float32),
    )(tokT)


def kernel(x, token_table, position_table):
    r = x.reshape(_ROWS).astype(jnp.int32)
    rp = (
        ((r >> 11) << 11)
        | ((r & (_TB // 2 - 1)) << 1)
        | ((r >> 10) & 1)
    )
    x_de = rp.reshape(_HALF, 2).T.reshape(_ROWS)
    tok64 = _relayout(token_table.T).reshape(_TROWS, _D)
    pos2 = position_table.reshape(_SEQ // 2, 2 * _D)
    out = _run(x_de, tok64, pos2)
    return out.reshape(_BATCH, _SEQ, _D)


# speedup vs baseline: 1.0230x; 1.0230x over previous
import functools

import jax
import jax.numpy as jnp
from jax import lax
from jax.experimental import pallas as pl
from jax.experimental.pallas import tpu as pltpu
from jax.experimental.pallas import tpu_sc as plsc

_VOCAB = 1000000
_D = 64
_SEQ = 200
_BATCH = 1024
_NC = 2
_NS = 16
_NW = _NC * _NS
_ROWS = _BATCH * _SEQ
_CHUNK = 2 * _SEQ
_PAIRS = _CHUNK // 2
_CHUNKS_PER_W = _ROWS // (_NW * _CHUNK)
_HALF = _ROWS // 2
_IDXW = _CHUNKS_PER_W * _PAIRS
_NBUF = 2
_LANES = 16
_VPR = _D // _LANES


def _emb_kernel(x_hbm, tok_hbm, pos2_hbm, out_hbm, pos_v, idx_raw,
                rows_e, rows_o, pout, sems):
    sem_g, sem_o = sems
    wid = lax.axis_index("s") * _NC + lax.axis_index("c")
    base = wid * 2 * _IDXW

    pltpu.sync_copy(pos2_hbm, pos_v)
    pltpu.sync_copy(x_hbm.at[pl.ds(base, 2 * _IDXW)], idx_raw)

    def stage(c):
        b = c % _NBUF
        he = pltpu.async_copy(
            tok_hbm.at[idx_raw.at[pl.ds(c * _CHUNK, _PAIRS)]],
            rows_e[b],
            sem_g[b],
        )
        ho = pltpu.async_copy(
            tok_hbm.at[idx_raw.at[pl.ds(c * _CHUNK + _PAIRS, _PAIRS)]],
            rows_o[b],
            sem_g[b],
        )
        return he, ho

    def merge_add(b):
        def pair(p, s2x100):
            p_abs = s2x100 + p
            for j in range(_VPR):
                pout[b][p_abs, pl.ds(j * _LANES, _LANES)] = (
                    rows_e[b][p_abs, pl.ds(j * _LANES, _LANES)]
                    + pos_v[p, pl.ds(j * _LANES, _LANES)]
                )
            for j in range(_VPR):
                pout[b][p_abs, pl.ds(_D + j * _LANES, _LANES)] = (
                    rows_o[b][p_abs, pl.ds(j * _LANES, _LANES)]
                    + pos_v[p, pl.ds(_D + j * _LANES, _LANES)]
                )
            return s2x100

        for s2 in range(2):
            lax.fori_loop(0, _PAIRS // 2, pair, s2 * (_PAIRS // 2), unroll=2)

    g = [None] * _CHUNKS_PER_W
    o = [None] * _CHUNKS_PER_W
    g[0] = stage(0)
    for c in range(_CHUNKS_PER_W):
        b = c % _NBUF
        if c + 1 < _CHUNKS_PER_W:
            g[c + 1] = stage(c + 1)
        g[c][0].wait()
        g[c][1].wait()
        if c >= _NBUF:
            o[c - _NBUF].wait()
        merge_add(b)
        o[c] = pltpu.async_copy(
            pout[b],
            out_hbm.at[pl.ds(base // 2 + c * _PAIRS, _PAIRS)],
            sem_o[b],
        )
    for c in range(_CHUNKS_PER_W - _NBUF, _CHUNKS_PER_W):
        o[c].wait()


@jax.jit
def _run(x_de, tok64, pos2):
    mesh = plsc.VectorSubcoreMesh(core_axis_name="c", subcore_axis_name="s")
    f = functools.partial(
        pl.kernel,
        out_type=jax.ShapeDtypeStruct((_ROWS // 2, 2 * _D), jnp.float32),
        mesh=mesh,
        scratch_types=[
            pltpu.VMEM((_SEQ // 2, 2 * _D), jnp.float32),
            pltpu.VMEM((2 * _IDXW,), jnp.int32),
            [pltpu.VMEM((_PAIRS, _D), jnp.float32) for _ in range(_NBUF)],
            [pltpu.VMEM((_PAIRS, _D), jnp.float32) for _ in range(_NBUF)],
            [pltpu.VMEM((_PAIRS, 2 * _D), jnp.float32) for _ in range(_NBUF)],
            (
                [pltpu.SemaphoreType.DMA for _ in range(_NBUF)],
                [pltpu.SemaphoreType.DMA for _ in range(_NBUF)],
            ),
        ],
        compiler_params=pltpu.CompilerParams(use_tc_tiling_on_sc=False),
    )(_emb_kernel)
    return f(x_de, tok64, pos2)


_TB = 2048
_GRID = (_VOCAB + _TB - 1) // _TB
_TROWS = _GRID * _TB


def _transpose_body(in_ref, out_ref):
    t = jnp.transpose(in_ref[...])
    out_ref[...] = jnp.concatenate([t[: _TB // 2], t[_TB // 2 :]], axis=1)


@jax.jit
def _relayout(tokT):
    return pl.pallas_call(
        _transpose_body,
        grid=(_GRID,),
        in_specs=[pl.BlockSpec((_D, _TB), lambda g: (0, g))],
        out_specs=pl.BlockSpec((_TB // 2, 2 * _D), lambda g: (g, 0)),
        out_shape=jax.ShapeDtypeStruct((_TROWS // 2, 2 * _D), jnp.float32),
    )(tokT)


def kernel(x, token_table, position_table):
    r = x.reshape(_ROWS).astype(jnp.int32)
    rp = ((r >> 11) << 11) | ((r & (_TB // 2 - 1)) << 1) | ((r >> 10) & 1)
    x3 = rp.reshape(_ROWS // _CHUNK, _CHUNK)
    x_flat = jnp.concatenate([x3[:, 0::2], x3[:, 1::2]], axis=1).reshape(_ROWS)
    tok64 = _relayout(token_table.T).reshape(_TROWS, _D)
    pos2 = position_table.reshape(_SEQ // 2, 2 * _D)
    out = _run(x_flat, tok64, pos2)
    return out.reshape(_BATCH, _SEQ, _D)
